# Initial kernel scaffold; baseline (speedup 1.0000x reference)
import jax
import jax.numpy as jnp
from jax.experimental import pallas as pl


def kernel(x, w1, b1, w2, b2, fc_wt, fc_b):
    raise NotImplementedError("write your pallas kernel here")


# speedup vs baseline: 3.1996x; 3.1996x over previous
import functools

import jax
import jax.numpy as jnp
from jax.experimental import pallas as pl
from jax.experimental.pallas import tpu as pltpu

K5 = 5
PAD = 2


def _sel(rows, cols, pred):
    i = jax.lax.broadcasted_iota(jnp.int32, (rows, cols), 0)
    j = jax.lax.broadcasted_iota(jnp.int32, (rows, cols), 1)
    return pred(i, j).astype(jnp.float32)


def _convnet_kernel(x_ref, w1_ref, b1_ref, w2_ref, b2_ref, o_ref, xp5, mid5, *,
                    cin, c1, c2, h, w):
    h2, w2d = h // 2, w // 2
    ho, wo = h // 4, w // 4
    mw = w2d + 2 * PAD
    rb = h // 2
    rbo = rb // 2

    xp5[...] = jnp.zeros_like(xp5)
    mid5[...] = jnp.zeros_like(mid5)

    sel_re = _sel(rbo, rb, lambda i, j: j == 2 * i)
    sel_ro = _sel(rbo, rb, lambda i, j: j == 2 * i + 1)
    sel_c1e = _sel(w, mw, lambda i, j: i == 2 * (j - PAD))
    sel_c1o = _sel(w, mw, lambda i, j: i == 2 * (j - PAD) + 1)
    sel_r2e = _sel(ho, h2, lambda i, j: j == 2 * i)
    sel_r2o = _sel(ho, h2, lambda i, j: j == 2 * i + 1)
    sel_c2e = _sel(2 * mw, 2 * wo,
                   lambda i, j: i == jnp.where(j < wo, 2 * j, 2 * (j - wo) + 2 * mw // 2))
    sel_c2o = _sel(2 * mw, 2 * wo,
                   lambda i, j: i == jnp.where(j < wo, 2 * j + 1, 2 * (j - wo) + 2 * mw // 2 + 1))

    for im in range(2):
        for dw in range(K5):
            lo = max(0, PAD - dw)
            hi = min(w, w + PAD - dw)
            xp5[dw, :, PAD:PAD + h, lo:hi] = x_ref[im, :, :, lo + dw - PAD:hi + dw - PAD]

        for s in range(h // rb):
            r0 = s * rb
            for g in range(2):
                accs = [None] * 4
                for ci in range(cin):
                    for dh in range(K5):
                        for dw in range(K5):
                            tap = xp5[dw, ci, r0 + dh:r0 + dh + rb, 0:w]
                            for c in range(4):
                                co = 4 * g + c
                                wv = w1_ref[((co * cin + ci) * K5 + dh) * K5 + dw]
                                t = tap * wv
                                accs[c] = t if accs[c] is None else accs[c] + t
                for c in range(4):
                    co = 4 * g + c
                    act = jnp.maximum(accs[c] + b1_ref[co], 0.0)
                    rows = jnp.maximum(
                        jnp.dot(sel_re, act, preferred_element_type=jnp.float32),
                        jnp.dot(sel_ro, act, preferred_element_type=jnp.float32))
                    pooled = jnp.maximum(
                        jnp.dot(rows, sel_c1e, preferred_element_type=jnp.float32),
                        jnp.dot(rows, sel_c1o, preferred_element_type=jnp.float32))
                    mid5[0, co, PAD + s * rbo:PAD + (s + 1) * rbo,
                         im * mw:(im + 1) * mw] = pooled

    for dw in range(1, K5):
        mid5[dw, :, :, 0:2 * mw - dw] = mid5[0, :, :, dw:2 * mw]

    for g in range(2):
        accs = [None] * 4
        for ci in range(c1):
            for dh in range(K5):
                for dw in range(K5):
                    tap = mid5[dw, ci, dh:dh + h2, 0:2 * mw]
                    for c in range(4):
                        co = 4 * g + c
                        wv = w2_ref[((co * c1 + ci) * K5 + dh) * K5 + dw]
                        t = tap * wv
                        accs[c] = t if accs[c] is None else accs[c] + t
        for c in range(4):
            co = 4 * g + c
            act = jnp.maximum(accs[c] + b2_ref[co], 0.0)
            rows = jnp.maximum(
                jnp.dot(sel_r2e, act, preferred_element_type=jnp.float32),
                jnp.dot(sel_r2o, act, preferred_element_type=jnp.float32))
            pooled = jnp.maximum(
                jnp.dot(rows, sel_c2e, preferred_element_type=jnp.float32),
                jnp.dot(rows, sel_c2o, preferred_element_type=jnp.float32))
            o_ref[0, co, :, :] = pooled[:, 0:wo]
            o_ref[1, co, :, :] = pooled[:, wo:2 * wo]


def _conv_layers(x, w1, b1, w2, b2):
    n, cin, h, w = x.shape
    c1 = w1.shape[0]
    c2 = w2.shape[0]
    assert n % 2 == 0 and h % 4 == 0 and w % 4 == 0, (n, h, w)

    kern = functools.partial(_convnet_kernel, cin=cin, c1=c1, c2=c2, h=h, w=w)
    return pl.pallas_call(
        kern,
        out_shape=jax.ShapeDtypeStruct((n, c2, h // 4, w // 4), jnp.float32),
        grid=(n // 2,),
        in_specs=[
            pl.BlockSpec((2, cin, h, w), lambda i: (i, 0, 0, 0)),
            pl.BlockSpec(memory_space=pltpu.MemorySpace.SMEM),
            pl.BlockSpec(memory_space=pltpu.MemorySpace.SMEM),
            pl.BlockSpec(memory_space=pltpu.MemorySpace.SMEM),
            pl.BlockSpec(memory_space=pltpu.MemorySpace.SMEM),
        ],
        out_specs=pl.BlockSpec((2, c2, h // 4, w // 4), lambda i: (i, 0, 0, 0)),
        scratch_shapes=[
            pltpu.VMEM((K5, cin, h + 4 * PAD, 128), jnp.float32),
            pltpu.VMEM((K5, c1, h // 2 + 4 * PAD, 128), jnp.float32),
        ],
        compiler_params=pltpu.CompilerParams(dimension_semantics=("parallel",)),
    )(x,
      w1.reshape(-1).astype(jnp.float32), b1.astype(jnp.float32),
      w2.reshape(-1).astype(jnp.float32), b2.astype(jnp.float32))


def _fc_kernel(a_ref, w_ref, b_ref, o_ref):
    o_ref[...] = (jnp.dot(a_ref[...], w_ref[...],
                          preferred_element_type=jnp.float32) + b_ref[...])


def _fc(a, w_t, b):
    m, k = a.shape
    k2, nf = w_t.shape
    assert k == k2
    return pl.pallas_call(
        _fc_kernel,
        out_shape=jax.ShapeDtypeStruct((m, nf), jnp.float32),
        grid=(2,),
        in_specs=[pl.BlockSpec((m // 2, k), lambda i: (i, 0)),
                  pl.BlockSpec((k, nf), lambda i: (0, 0)),
                  pl.BlockSpec((1, nf), lambda i: (0, 0))],
        out_specs=pl.BlockSpec((m // 2, nf), lambda i: (i, 0)),
        compiler_params=pltpu.CompilerParams(dimension_semantics=("parallel",)),
    )(a, w_t, b.reshape(1, nf))


def kernel(x, w1, b1, w2, b2, fc_wt, fc_b):
    y = _conv_layers(x, w1, b1, w2, b2)
    flat = y.reshape(y.shape[0], -1)
    return _fc(flat, fc_wt, fc_b)
